# baseline (speedup 1.0000x reference)
import functools

import jax
import jax.numpy as jnp
from jax.experimental import pallas as pl

MAX_DEPTH = 8
H = 128
EDGE = 16
B = 2000


def _mlp(inp, W1, b1, W2, b2):
    h = jnp.maximum(jnp.dot(inp, W1, preferred_element_type=jnp.float32) + b1, 0.0)
    return jnp.dot(h, W2, preferred_element_type=jnp.float32) + b2


def _processed_body(x_ref, plef_ref, W1p_ref, b1p_ref, W2p_ref, b2p_ref, out_ref):
    inp = jnp.concatenate([x_ref[...], plef_ref[...]], axis=1)
    out_ref[...] = _mlp(inp, W1p_ref[...], b1p_ref[...], W2p_ref[...], b2p_ref[...])


def _update_body(x_ref, left_ref, right_ref, pef_ref, mh_ref, pcnt_ref, dcnt_ref,
                 W1m_ref, b1m_ref, W2m_ref, b2m_ref,
                 W1e_ref, b1e_ref, W2e_ref, b2e_ref, out_ref):
    x = x_ref[...]
    pmask = pcnt_ref[...].astype(jnp.int32) != 0
    dmask = dcnt_ref[...].astype(jnp.int32) != 0
    inp_m = jnp.concatenate([left_ref[...], right_ref[...], pef_ref[...]], axis=1)
    x_parents = _mlp(inp_m, W1m_ref[...], b1m_ref[...], W2m_ref[...], b2m_ref[...])
    x_desig = jnp.where(dmask, x, 0.0)
    inp_e = jnp.concatenate([x_desig, mh_ref[...]], axis=1)
    x_merged = _mlp(inp_e, W1e_ref[...], b1e_ref[...], W2e_ref[...], b2e_ref[...])
    out_ref[...] = jnp.where(pmask, x_parents, jnp.where(dmask, x_merged, x))


def _row_spec(w):
    return pl.BlockSpec((B, w), lambda i: (i, 0))


def _full_spec(shape):
    nd = len(shape)
    return pl.BlockSpec(shape, (lambda i: (0,) * nd))


def kernel(x, parent_edge_features, parent_light_edge_features, edge_index, depths, states,
           W1m, b1m, W2m, b2m, W1p, b1p, W2p, b2p, W1e, b1e, W2e, b2e):
    n = x.shape[0]
    grid = (n // B,)
    b1m2 = b1m.reshape(1, H)
    b2m2 = b2m.reshape(1, H)
    b1p2 = b1p.reshape(1, H)
    b2p2 = b2p.reshape(1, H)
    b1e2 = b1e.reshape(1, H)
    b2e2 = b2e.reshape(1, H)

    processed_call = pl.pallas_call(
        _processed_body,
        grid=grid,
        in_specs=[_row_spec(H), _row_spec(EDGE),
                  _full_spec(W1p.shape), _full_spec(b1p2.shape),
                  _full_spec(W2p.shape), _full_spec(b2p2.shape)],
        out_specs=_row_spec(H),
        out_shape=jax.ShapeDtypeStruct((n, H), jnp.float32),
    )
    update_call = pl.pallas_call(
        _update_body,
        grid=grid,
        in_specs=[_row_spec(H), _row_spec(H), _row_spec(H), _row_spec(EDGE),
                  _row_spec(H), _row_spec(1), _row_spec(1),
                  _full_spec(W1m.shape), _full_spec(b1m2.shape),
                  _full_spec(W2m.shape), _full_spec(b2m2.shape),
                  _full_spec(W1e.shape), _full_spec(b1e2.shape),
                  _full_spec(W2e.shape), _full_spec(b2e2.shape)],
        out_specs=_row_spec(H),
        out_shape=jax.ShapeDtypeStruct((n, H), jnp.float32),
    )

    parents = jnp.zeros((n,), dtype=edge_index.dtype).at[edge_index[0]].set(edge_index[1])
    ones_col = jnp.ones((n, 1), dtype=x.dtype)
    zeros_col = jnp.zeros((n, 1), dtype=x.dtype)
    for depth in range(MAX_DEPTH - 1, 0, -1):
        mask_depth = depths == depth
        left_mask = mask_depth & (states == 0)
        right_mask = mask_depth & (states == 1)
        heads_mask = mask_depth & (states == 3)
        left = jnp.zeros_like(x).at[parents].add(jnp.where(left_mask[:, None], x, 0.0))
        right = jnp.zeros_like(x).at[parents].add(jnp.where(right_mask[:, None], x, 0.0))
        parents_cnt = zeros_col.at[parents].add(jnp.where(left_mask[:, None], ones_col, 0.0))
        processed_all = processed_call(x, parent_light_edge_features, W1p, b1p2, W2p, b2p2)
        merged_heads = jnp.zeros_like(x).at[parents].add(
            jnp.where(heads_mask[:, None], processed_all, 0.0))
        designated_cnt = zeros_col.at[parents].add(jnp.where(heads_mask[:, None], ones_col, 0.0))
        x = update_call(x, left, right, parent_edge_features, merged_heads,
                        parents_cnt, designated_cnt,
                        W1m, b1m2, W2m, b2m2, W1e, b1e2, W2e, b2e2)
    return x
